# baseline (speedup 1.0000x reference)
import functools

import jax
import jax.numpy as jnp
from jax import lax
from jax.experimental import pallas as pl
from jax.experimental.pallas import tpu as pltpu
from jax.experimental.pallas import tpu_sc as plsc

THRESH = 0.5
B = 256
S7 = 7
HW = 49
C = 512
K = 1000

NC = 2
NS = 16
NW = NC * NS
BPW = B // NW

APAD = 64
LANES = 16
LTILE = 128


def _sc_argmax_gather(branchA, interm):
    mesh = plsc.VectorSubcoreMesh(
        core_axis_name="c", subcore_axis_name="s",
        num_cores=NC, num_subcores=NS)

    @functools.partial(
        pl.kernel,
        out_type=jax.ShapeDtypeStruct((B * APAD,), jnp.float32),
        mesh=mesh,
        compiler_params=pltpu.CompilerParams(
            needs_layout_passes=False, use_tc_tiling_on_sc=False),
        scratch_types=[
            pltpu.VMEM((BPW, K), jnp.float32),
            pltpu.VMEM((BPW, S7, S7, LTILE), jnp.float32),
            pltpu.VMEM((BPW * APAD,), jnp.float32),
            pltpu.VMEM((LANES,), jnp.float32),
            pltpu.VMEM((LANES,), jnp.int32),
            pltpu.SemaphoreType.DMA,
        ],
    )
    def sc_k(branchA_hbm, interm_hbm, out_hbm, rowbuf, colbuf, valbuf,
             tmpv, tmpi, sem):
        wid = lax.axis_index("s") * NC + lax.axis_index("c")
        base = wid * BPW
        pltpu.sync_copy(branchA_hbm.at[pl.ds(base, BPW), :], rowbuf)
        lanes = lax.iota(jnp.int32, LANES)
        copies = []
        lane_ids = []
        for s in range(BPW):
            def body(c, carry):
                bv, bi = carry
                v = rowbuf[s, pl.ds(c * LANES, LANES)]
                g = c * LANES + lanes
                take = v > bv
                return jnp.where(take, v, bv), jnp.where(take, g, bi)

            init = (jnp.full((LANES,), -jnp.inf, jnp.float32),
                    jnp.zeros((LANES,), jnp.int32))
            bv, bi = lax.fori_loop(0, K // LANES, body, init)
            v = rowbuf[s, pl.ds(K - LANES, LANES)]
            g = (K - LANES) + lanes
            take = v > bv
            bv = jnp.where(take, v, bv)
            bi = jnp.where(take, g, bi)
            for sh in (8, 4, 2, 1):
                tmpv[...] = bv
                tmpi[...] = bi
                perm = jnp.bitwise_xor(lanes, sh)
                pv = plsc.load_gather(tmpv, [perm])
                pi = plsc.load_gather(tmpi, [perm])
                better = (pv > bv) | ((pv == bv) & (pi < bi))
                bv = jnp.where(better, pv, bv)
                bi = jnp.where(better, pi, bi)
            idx = bi[0]
            tile = lax.div(idx, jnp.int32(LTILE))
            lane_ids.append(idx - tile * LTILE)
            copies.append(pltpu.async_copy(
                interm_hbm.at[base + s, :, :,
                              pl.ds(pl.multiple_of(tile * LTILE, LTILE),
                                    LTILE)],
                colbuf.at[s], sem))
        for s in range(BPW):
            copies[s].wait()
            lane_vec = jnp.zeros((LANES,), jnp.int32) + lane_ids[s]
            for j in range(APAD // LANES):
                p = jnp.minimum(j * LANES + lanes, HW - 1)
                val = plsc.load_gather(
                    colbuf.at[s], [p // S7, p % S7, lane_vec])
                valbuf[pl.ds(s * APAD + j * LANES, LANES)] = val
        pltpu.sync_copy(valbuf, out_hbm.at[pl.ds(base * APAD, BPW * APAD)])

    return sc_k(branchA, interm)


def _tc_apply(vgg_end, a4):
    BC = 8

    def body(vgg_ref, a_ref, out_ref):
        a = a_ref[...]
        m = jnp.where(a > THRESH, a, jnp.zeros_like(a))
        out_ref[...] = vgg_ref[...] - m

    return pl.pallas_call(
        body,
        grid=(B // BC,),
        in_specs=[
            pl.BlockSpec((BC, S7, S7, C), lambda i: (i, 0, 0, 0)),
            pl.BlockSpec((BC, S7, S7, 1), lambda i: (i, 0, 0, 0)),
        ],
        out_specs=pl.BlockSpec((BC, S7, S7, C), lambda i: (i, 0, 0, 0)),
        out_shape=jax.ShapeDtypeStruct((B, S7, S7, C), jnp.float32),
    )(vgg_end, a4)


def kernel(vgg_end, interm, branchA_end):
    a = _sc_argmax_gather(branchA_end, interm)
    a4 = a.reshape(B, APAD)[:, :HW].reshape(B, S7, S7, 1)
    return _tc_apply(vgg_end, a4)


# speedup vs baseline: 3.6783x; 3.6783x over previous
import functools

import jax
import jax.numpy as jnp
from jax import lax
from jax.experimental import pallas as pl
from jax.experimental.pallas import tpu as pltpu
from jax.experimental.pallas import tpu_sc as plsc

THRESH = 0.5
B = 256
S7 = 7
HW = 49
C = 512
K = 1000

NC = 2
NS = 16
NW = NC * NS
BPW = B // NW

LANES = 16
APAD = 64
BC = 32

QT = K // 8
BT = B // 128
TW = 8 * 128
PSTRIDE = QT * BT * TW


def _sc_argmax_gather(branchA, intermv):
    mesh = plsc.VectorSubcoreMesh(
        core_axis_name="c", subcore_axis_name="s",
        num_cores=NC, num_subcores=NS)

    @functools.partial(
        pl.kernel,
        out_type=jax.ShapeDtypeStruct((B * APAD,), jnp.float32),
        mesh=mesh,
        compiler_params=pltpu.CompilerParams(
            needs_layout_passes=False, use_tc_tiling_on_sc=False),
        scratch_types=[
            pltpu.VMEM((BPW, K), jnp.float32),
            pltpu.VMEM((BPW, APAD), jnp.int32),
            pltpu.VMEM((BPW * APAD,), jnp.float32),
            pltpu.VMEM((LANES,), jnp.float32),
            pltpu.VMEM((LANES,), jnp.int32),
            pltpu.SemaphoreType.DMA,
        ],
    )
    def sc_k(branchA_hbm, intermv_hbm, out_hbm, rowbuf, idxbuf, valbuf,
             tmpv, tmpi, sem):
        wid = lax.axis_index("s") * NC + lax.axis_index("c")
        base = wid * BPW
        pltpu.sync_copy(branchA_hbm.at[pl.ds(base, BPW), :], rowbuf)
        lanes = lax.iota(jnp.int32, LANES)
        copies = []
        for s in range(BPW):
            def body(c, carry):
                bv, bi = carry
                v = rowbuf[s, pl.ds(c * LANES, LANES)]
                g = c * LANES + lanes
                take = v > bv
                return jnp.where(take, v, bv), jnp.where(take, g, bi)

            init = (jnp.full((LANES,), -jnp.inf, jnp.float32),
                    jnp.zeros((LANES,), jnp.int32))
            bv, bi = lax.fori_loop(0, K // LANES, body, init)
            v = rowbuf[s, pl.ds(K - LANES, LANES)]
            g = (K - LANES) + lanes
            take = v > bv
            bv = jnp.where(take, v, bv)
            bi = jnp.where(take, g, bi)
            for sh in (8, 4, 2, 1):
                tmpv[...] = bv
                tmpi[...] = bi
                perm = jnp.bitwise_xor(lanes, sh)
                pv = plsc.load_gather(tmpv, [perm])
                pi = plsc.load_gather(tmpi, [perm])
                better = (pv > bv) | ((pv == bv) & (pi < bi))
                bv = jnp.where(better, pv, bv)
                bi = jnp.where(better, pi, bi)
            b = base + s
            bt = lax.div(b, jnp.int32(128))
            bl = b - bt * 128
            q = lax.div(bi, jnp.int32(8))
            r = bi - q * 8
            w0 = (q * BT + bt) * TW + r * 128 + bl
            for j in range(APAD // LANES):
                p = jnp.minimum(j * LANES + lanes, HW - 1)
                idxbuf[s, pl.ds(j * LANES, LANES)] = w0 + p * PSTRIDE
            copies.append(pltpu.async_copy(
                intermv_hbm.at[idxbuf.at[s]],
                valbuf.at[pl.ds(s * APAD, APAD)], sem))
        for c in copies:
            c.wait()
        pltpu.sync_copy(valbuf, out_hbm.at[pl.ds(base * APAD, BPW * APAD)])

    return sc_k(branchA, intermv)


def _tc_apply(vgg_t, a_t):
    def body(vgg_ref, a_ref, out_ref):
        a = a_ref[...]
        m = jnp.where(a > THRESH, a, jnp.zeros_like(a))
        out_ref[...] = vgg_ref[...] - m

    return pl.pallas_call(
        body,
        grid=(B // BC,),
        in_specs=[
            pl.BlockSpec((S7, S7, BC, C), lambda ib: (0, 0, ib, 0)),
            pl.BlockSpec((S7, S7, BC, 1), lambda ib: (0, 0, ib, 0)),
        ],
        out_specs=pl.BlockSpec((S7, S7, BC, C), lambda ib: (0, 0, ib, 0)),
        out_shape=jax.ShapeDtypeStruct((S7, S7, B, C), jnp.float32),
    )(vgg_t, a_t)


def kernel(vgg_end, interm, branchA_end):
    intermv = (interm.transpose(1, 2, 3, 0)
               .reshape(HW, QT, 8, BT, 128)
               .transpose(0, 1, 3, 2, 4)
               .reshape(B * HW * K))
    vgg_t = vgg_end.transpose(1, 2, 0, 3)
    a = _sc_argmax_gather(branchA_end, intermv)
    a_t = a.reshape(B, APAD)[:, :HW].transpose(1, 0).reshape(S7, S7, B, 1)
    out_t = _tc_apply(vgg_t, a_t)
    return out_t.transpose(2, 0, 1, 3)


# speedup vs baseline: 4.4891x; 1.2204x over previous
import functools

import jax
import jax.numpy as jnp
from jax import lax
from jax.experimental import pallas as pl
from jax.experimental.pallas import tpu as pltpu
from jax.experimental.pallas import tpu_sc as plsc

THRESH = 0.5
B = 256
S7 = 7
HW = 49
C = 512
K = 1000

NC = 2
NS = 16
NW = NC * NS
BPW = B // NW

LANES = 16
APAD = 64
BC = 32

QT = K // 8
BT = B // 128
TW = 8 * 128
PSTRIDE = QT * BT * TW


def _sc_argmax_gather(branchA, intermv):
    mesh = plsc.VectorSubcoreMesh(
        core_axis_name="c", subcore_axis_name="s",
        num_cores=NC, num_subcores=NS)

    @functools.partial(
        pl.kernel,
        out_type=jax.ShapeDtypeStruct((B * APAD,), jnp.float32),
        mesh=mesh,
        compiler_params=pltpu.CompilerParams(
            needs_layout_passes=False, use_tc_tiling_on_sc=False),
        scratch_types=[
            pltpu.VMEM((BPW, K), jnp.float32),
            pltpu.VMEM((BPW, APAD), jnp.int32),
            pltpu.VMEM((BPW * APAD,), jnp.float32),
            pltpu.VMEM((LANES,), jnp.float32),
            pltpu.VMEM((LANES,), jnp.int32),
            pltpu.SemaphoreType.DMA,
        ],
    )
    def sc_k(branchA_hbm, intermv_hbm, out_hbm, rowbuf, idxbuf, valbuf,
             tmpv, tmpi, sem):
        wid = lax.axis_index("s") * NC + lax.axis_index("c")
        base = wid * BPW
        pltpu.sync_copy(branchA_hbm.at[pl.ds(base, BPW), :], rowbuf)
        lanes = lax.iota(jnp.int32, LANES)
        copies = []
        for s in range(BPW):
            def body(c, carry):
                bv, bi = carry
                v = rowbuf[s, pl.ds(c * LANES, LANES)]
                g = c * LANES + lanes
                take = v > bv
                return jnp.where(take, v, bv), jnp.where(take, g, bi)

            init = (jnp.full((LANES,), -jnp.inf, jnp.float32),
                    jnp.zeros((LANES,), jnp.int32))
            bv, bi = lax.fori_loop(0, K // LANES, body, init)
            v = rowbuf[s, pl.ds(K - LANES, LANES)]
            g = (K - LANES) + lanes
            take = v > bv
            bv = jnp.where(take, v, bv)
            bi = jnp.where(take, g, bi)
            for sh in (8, 4, 2, 1):
                tmpv[...] = bv
                tmpi[...] = bi
                perm = jnp.bitwise_xor(lanes, sh)
                pv = plsc.load_gather(tmpv, [perm])
                pi = plsc.load_gather(tmpi, [perm])
                better = (pv > bv) | ((pv == bv) & (pi < bi))
                bv = jnp.where(better, pv, bv)
                bi = jnp.where(better, pi, bi)
            b = base + s
            bt = lax.div(b, jnp.int32(128))
            bl = b - bt * 128
            q = lax.div(bi, jnp.int32(8))
            r = bi - q * 8
            w0 = (q * BT + bt) * TW + r * 128 + bl
            for j in range(APAD // LANES):
                p = jnp.minimum(j * LANES + lanes, HW - 1)
                idxbuf[s, pl.ds(j * LANES, LANES)] = w0 + p * PSTRIDE
            copies.append(pltpu.async_copy(
                intermv_hbm.at[idxbuf.at[s]],
                valbuf.at[pl.ds(s * APAD, APAD)], sem))
        for c in copies:
            c.wait()
        pltpu.sync_copy(valbuf, out_hbm.at[pl.ds(base * APAD, BPW * APAD)])

    return sc_k(branchA, intermv)


def _tc_apply(vgg_t, a2):
    def body(vgg_ref, a_ref, out_ref):
        a = a_ref[...][:, :HW]
        at = jnp.transpose(a, (1, 0))
        m = jnp.where(at > THRESH, at, jnp.zeros_like(at))
        m4 = m.reshape(S7, S7, BC)[:, :, :, None]
        out_ref[...] = vgg_ref[...] - m4

    return pl.pallas_call(
        body,
        grid=(B // BC,),
        in_specs=[
            pl.BlockSpec((S7, S7, BC, C), lambda ib: (0, 0, ib, 0)),
            pl.BlockSpec((BC, APAD), lambda ib: (ib, 0)),
        ],
        out_specs=pl.BlockSpec((S7, S7, BC, C), lambda ib: (0, 0, ib, 0)),
        out_shape=jax.ShapeDtypeStruct((S7, S7, B, C), jnp.float32),
    )(vgg_t, a2)


def kernel(vgg_end, interm, branchA_end):
    intermv = (interm.transpose(1, 2, 3, 0)
               .reshape(HW, QT, 8, BT, 128)
               .transpose(0, 1, 3, 2, 4)
               .reshape(B * HW * K))
    vgg_t = vgg_end.transpose(1, 2, 0, 3)
    a = _sc_argmax_gather(branchA_end, intermv)
    out_t = _tc_apply(vgg_t, a.reshape(B, APAD))
    return out_t.transpose(2, 0, 1, 3)


# speedup vs baseline: 4.6545x; 1.0368x over previous
import functools

import jax
import jax.numpy as jnp
from jax import lax
from jax.experimental import pallas as pl
from jax.experimental.pallas import tpu as pltpu
from jax.experimental.pallas import tpu_sc as plsc

THRESH = 0.5
B = 256
S7 = 7
HW = 49
C = 512
K = 1000

NC = 2
NS = 16
NW = NC * NS
BPW = B // NW

LANES = 16
APAD = 64
BC = 64

QT = K // 8
BT = B // 128
TW = 8 * 128
PSTRIDE = QT * BT * TW


def _sc_argmax_gather(branchA, intermv):
    mesh = plsc.VectorSubcoreMesh(
        core_axis_name="c", subcore_axis_name="s",
        num_cores=NC, num_subcores=NS)

    @functools.partial(
        pl.kernel,
        out_type=jax.ShapeDtypeStruct((B * APAD,), jnp.float32),
        mesh=mesh,
        compiler_params=pltpu.CompilerParams(
            needs_layout_passes=False, use_tc_tiling_on_sc=False),
        scratch_types=[
            pltpu.VMEM((BPW, K), jnp.float32),
            pltpu.VMEM((BPW, APAD), jnp.int32),
            pltpu.VMEM((BPW * APAD,), jnp.float32),
            pltpu.VMEM((LANES,), jnp.float32),
            pltpu.VMEM((LANES,), jnp.int32),
            pltpu.SemaphoreType.DMA,
        ],
    )
    def sc_k(branchA_hbm, intermv_hbm, out_hbm, rowbuf, idxbuf, valbuf,
             tmpv, tmpi, sem):
        wid = lax.axis_index("s") * NC + lax.axis_index("c")
        base = wid * BPW
        pltpu.sync_copy(branchA_hbm.at[pl.ds(base, BPW), :], rowbuf)
        lanes = lax.iota(jnp.int32, LANES)
        copies = []
        for s in range(BPW):
            def body(c, carry):
                bv, bi = carry
                v = rowbuf[s, pl.ds(c * LANES, LANES)]
                g = c * LANES + lanes
                take = v > bv
                return jnp.where(take, v, bv), jnp.where(take, g, bi)

            init = (jnp.full((LANES,), -jnp.inf, jnp.float32),
                    jnp.zeros((LANES,), jnp.int32))
            bv, bi = lax.fori_loop(0, K // LANES, body, init)
            v = rowbuf[s, pl.ds(K - LANES, LANES)]
            g = (K - LANES) + lanes
            take = v > bv
            bv = jnp.where(take, v, bv)
            bi = jnp.where(take, g, bi)
            for sh in (8, 4, 2, 1):
                tmpv[...] = bv
                tmpi[...] = bi
                perm = jnp.bitwise_xor(lanes, sh)
                pv = plsc.load_gather(tmpv, [perm])
                pi = plsc.load_gather(tmpi, [perm])
                better = (pv > bv) | ((pv == bv) & (pi < bi))
                bv = jnp.where(better, pv, bv)
                bi = jnp.where(better, pi, bi)
            b = base + s
            bt = lax.div(b, jnp.int32(128))
            bl = b - bt * 128
            q = lax.div(bi, jnp.int32(8))
            r = bi - q * 8
            w0 = (q * BT + bt) * TW + r * 128 + bl
            for j in range(APAD // LANES):
                p = jnp.minimum(j * LANES + lanes, HW - 1)
                idxbuf[s, pl.ds(j * LANES, LANES)] = w0 + p * PSTRIDE
            copies.append(pltpu.async_copy(
                intermv_hbm.at[idxbuf.at[s]],
                valbuf.at[pl.ds(s * APAD, APAD)], sem))
        for c in copies:
            c.wait()
        pltpu.sync_copy(valbuf, out_hbm.at[pl.ds(base * APAD, BPW * APAD)])

    return sc_k(branchA, intermv)


def _tc_apply(vgg_t, a2):
    def body(vgg_ref, a_ref, out_ref):
        a = a_ref[...][:, :HW]
        at = jnp.transpose(a, (1, 0))
        m = jnp.where(at > THRESH, at, jnp.zeros_like(at))
        m4 = m.reshape(S7, S7, BC)[:, :, :, None]
        out_ref[...] = vgg_ref[...] - m4

    return pl.pallas_call(
        body,
        grid=(B // BC,),
        in_specs=[
            pl.BlockSpec((S7, S7, BC, C), lambda ib: (0, 0, ib, 0)),
            pl.BlockSpec((BC, APAD), lambda ib: (ib, 0)),
        ],
        out_specs=pl.BlockSpec((S7, S7, BC, C), lambda ib: (0, 0, ib, 0)),
        out_shape=jax.ShapeDtypeStruct((S7, S7, B, C), jnp.float32),
    )(vgg_t, a2)


def kernel(vgg_end, interm, branchA_end):
    intermv = (interm.transpose(1, 2, 3, 0)
               .reshape(HW, QT, 8, BT, 128)
               .transpose(0, 1, 3, 2, 4)
               .reshape(B * HW * K))
    vgg_t = vgg_end.transpose(1, 2, 0, 3)
    a = _sc_argmax_gather(branchA_end, intermv)
    out_t = _tc_apply(vgg_t, a.reshape(B, APAD))
    return out_t.transpose(2, 0, 1, 3)
